# Initial kernel scaffold; baseline (speedup 1.0000x reference)
import jax
import jax.numpy as jnp
from jax.experimental import pallas as pl


def kernel(x, mus, stdevs, weights):
    raise NotImplementedError("write your pallas kernel here")


# speedup vs baseline: 8.0480x; 8.0480x over previous
import math

import jax
import jax.numpy as jnp
from jax.experimental import pallas as pl

_N, _K, _D = 16384, 512, 16
_TILE_N = 2048


def _gmm_tile_kernel(xt_ref, must_ref, stdt_ref, w_ref, out_ref):
    xt = xt_ref[...]
    must = must_ref[...]
    stdt = stdt_ref[...]
    w = w_ref[...]

    log_std = jnp.log(stdt + 1e-12)
    iv = jnp.exp(-2.0 * log_std)

    m_w = jnp.max(w)
    lse_w = m_w + jnp.log(jnp.sum(jnp.exp(w - m_w)))
    c = (-0.5 * jnp.sum(must * must * iv, axis=0, keepdims=True)
         - jnp.sum(log_std, axis=0, keepdims=True)
         + w
         - 0.5 * _D * math.log(2.0 * math.pi)
         - lse_w)

    xa = jnp.concatenate([xt * xt, xt], axis=0)
    pa = jnp.concatenate([-0.5 * iv, must * iv], axis=0)
    logits = jax.lax.dot_general(
        pa, xa, (((0,), (0,)), ((), ())),
        preferred_element_type=jnp.float32,
        precision=jax.lax.Precision.HIGHEST) + c.reshape(_K, 1)

    m = jnp.max(logits, axis=0, keepdims=True)
    lse = m + jnp.log(jnp.sum(jnp.exp(logits - m), axis=0, keepdims=True))
    out_ref[...] = lse[None, :, :]


def kernel(x, mus, stdevs, weights):
    xt = x.T
    must = mus.T
    stdt = stdevs.T
    w2 = weights.reshape(1, _K)
    grid = (_N // _TILE_N,)
    out = pl.pallas_call(
        _gmm_tile_kernel,
        grid=grid,
        in_specs=[
            pl.BlockSpec((_D, _TILE_N), lambda i: (0, i)),
            pl.BlockSpec((_D, _K), lambda i: (0, 0)),
            pl.BlockSpec((_D, _K), lambda i: (0, 0)),
            pl.BlockSpec((1, _K), lambda i: (0, 0)),
        ],
        out_specs=pl.BlockSpec((1, 1, _TILE_N), lambda i: (i, 0, 0)),
        out_shape=jax.ShapeDtypeStruct((_N // _TILE_N, 1, _TILE_N), jnp.float32),
    )(xt, must, stdt, w2)
    return out.reshape(_N)


# speedup vs baseline: 8.0587x; 1.0013x over previous
import math

import jax
import jax.numpy as jnp
from jax.experimental import pallas as pl
from jax.experimental.pallas import tpu as pltpu

_N, _K, _D = 16384, 512, 16
_TILE_N = 2048


def _gmm_tile_kernel(xt_ref, must_ref, stdt_ref, w_ref, out_ref):
    xt = xt_ref[...]
    must = must_ref[...]
    stdt = stdt_ref[...]
    w = w_ref[...]

    log_std = jnp.log(stdt + 1e-12)
    iv = jnp.exp(-2.0 * log_std)

    m_w = jnp.max(w)
    lse_w = m_w + jnp.log(jnp.sum(jnp.exp(w - m_w)))
    c = (-0.5 * jnp.sum(must * must * iv, axis=0, keepdims=True)
         - jnp.sum(log_std, axis=0, keepdims=True)
         + w
         - 0.5 * _D * math.log(2.0 * math.pi)
         - lse_w)

    xa = jnp.concatenate([xt * xt, xt], axis=0)
    pa = jnp.concatenate([-0.5 * iv, must * iv], axis=0)
    logits = jax.lax.dot_general(
        pa, xa, (((0,), (0,)), ((), ())),
        preferred_element_type=jnp.float32,
        precision=jax.lax.Precision.HIGHEST) + c.reshape(_K, 1)

    m = jnp.max(logits, axis=0, keepdims=True)
    lse = m + jnp.log(jnp.sum(jnp.exp(logits - m), axis=0, keepdims=True))
    out_ref[...] = lse[None, :, :]


def kernel(x, mus, stdevs, weights):
    xt = x.T
    must = mus.T
    stdt = stdevs.T
    w2 = weights.reshape(1, _K)
    grid = (_N // _TILE_N,)
    out = pl.pallas_call(
        _gmm_tile_kernel,
        grid=grid,
        in_specs=[
            pl.BlockSpec((_D, _TILE_N), lambda i: (0, i)),
            pl.BlockSpec((_D, _K), lambda i: (0, 0)),
            pl.BlockSpec((_D, _K), lambda i: (0, 0)),
            pl.BlockSpec((1, _K), lambda i: (0, 0)),
        ],
        out_specs=pl.BlockSpec((1, 1, _TILE_N), lambda i: (i, 0, 0)),
        out_shape=jax.ShapeDtypeStruct((_N // _TILE_N, 1, _TILE_N), jnp.float32),
        compiler_params=pltpu.CompilerParams(
            dimension_semantics=("parallel",)),
    )(xt, must, stdt, w2)
    return out.reshape(_N)


# speedup vs baseline: 17.0859x; 2.1202x over previous
import math

import jax
import jax.numpy as jnp
from jax.experimental import pallas as pl
from jax.experimental.pallas import tpu as pltpu

_N, _K, _D = 16384, 512, 16
_TILE_N = 2048


def _gmm_tile_kernel(xt_ref, must_ref, stdt_ref, w_ref, out_ref):
    xt = xt_ref[...]
    must = must_ref[...]
    stdt = stdt_ref[...]
    w = w_ref[...]

    log_std = jnp.log(stdt + 1e-12)
    iv = jnp.exp(-2.0 * log_std)

    m_w = jnp.max(w)
    lse_w = m_w + jnp.log(jnp.sum(jnp.exp(w - m_w)))
    c = (-0.5 * jnp.sum(must * must * iv, axis=0, keepdims=True)
         - jnp.sum(log_std, axis=0, keepdims=True)
         + w
         - 0.5 * _D * math.log(2.0 * math.pi)
         - lse_w)

    xa = jnp.concatenate([xt * xt, xt], axis=0)
    pa = jnp.concatenate([-0.5 * iv, must * iv], axis=0)

    def limbs(a):
        hi = a.astype(jnp.bfloat16)
        r = a - hi.astype(jnp.float32)
        mid = r.astype(jnp.bfloat16)
        lo = (r - mid.astype(jnp.float32)).astype(jnp.bfloat16)
        return hi, mid, lo

    ph, pm, plo = limbs(pa)
    xh, xm, xl = limbs(xa)
    pcat = jnp.concatenate([ph, ph, pm, ph, pm, plo], axis=0)
    xcat = jnp.concatenate([xh, xm, xh, xl, xm, xh], axis=0)
    logits = jax.lax.dot_general(
        pcat, xcat, (((0,), (0,)), ((), ())),
        preferred_element_type=jnp.float32) + c.reshape(_K, 1)

    m = jnp.max(logits, axis=0, keepdims=True)
    lse = m + jnp.log(jnp.sum(jnp.exp(logits - m), axis=0, keepdims=True))
    out_ref[...] = lse[None, :, :]


def kernel(x, mus, stdevs, weights):
    xt = x.T
    must = mus.T
    stdt = stdevs.T
    w2 = weights.reshape(1, _K)
    grid = (_N // _TILE_N,)
    out = pl.pallas_call(
        _gmm_tile_kernel,
        grid=grid,
        in_specs=[
            pl.BlockSpec((_D, _TILE_N), lambda i: (0, i)),
            pl.BlockSpec((_D, _K), lambda i: (0, 0)),
            pl.BlockSpec((_D, _K), lambda i: (0, 0)),
            pl.BlockSpec((1, _K), lambda i: (0, 0)),
        ],
        out_specs=pl.BlockSpec((1, 1, _TILE_N), lambda i: (i, 0, 0)),
        out_shape=jax.ShapeDtypeStruct((_N // _TILE_N, 1, _TILE_N), jnp.float32),
        compiler_params=pltpu.CompilerParams(
            dimension_semantics=("parallel",)),
    )(xt, must, stdt, w2)
    return out.reshape(_N)
